# Initial kernel scaffold; baseline (speedup 1.0000x reference)
import jax
import jax.numpy as jnp
from jax.experimental import pallas as pl


def kernel(x_author, x_paper, edge_index_m0, edge_index_m1, edge_index_m2, edge_feat_m0, edge_feat_m1, edge_feat_m2, Wn0, Ws0, We0, b0, Wn1, Ws1, We1, b1, Wn2, Ws2, We2, b2):
    raise NotImplementedError("write your pallas kernel here")


# speedup vs baseline: 2.3737x; 2.3737x over previous
import jax
import jax.numpy as jnp
from jax import lax
from jax.experimental import pallas as pl
from jax.experimental.pallas import tpu as pltpu, tpu_sc as plsc

N_NODE = 10000
D_IN = 128
D_OUT = 128
D_EDGE = 16
E = 320000

NC = 2
NS = 16
NW = NC * NS
EW = E // NW
CHUNK = 80
NCHUNK = EW // CHUNK
NPAD = 10240
RPT = NPAD // NS
ZROWS = 80
ZITER = RPT // ZROWS


def _scx_body(xa, xp, src0, dst0, src1, dst1, src2, dst2,
              sx_out, accx, srcb, dstb, rows, sem):
    c = lax.axis_index("c")
    s = lax.axis_index("s")
    wid = c * NS + s
    zero16 = jnp.zeros((16,), jnp.float32)
    r0 = s * RPT
    o0 = c * NPAD + s * RPT

    def _one_type(m, src_h, dst_h, x_h):
        def _fill(i, _):
            for j in range(D_IN // 16):
                rows[i, pl.ds(j * 16, 16)] = zero16
            return 0

        lax.fori_loop(0, ZROWS, _fill, 0)

        @pl.loop(0, ZITER)
        def _zero(k):
            pltpu.async_copy(rows, accx.at[pl.ds(r0 + k * ZROWS, ZROWS)],
                             sem).wait()

        plsc.subcore_barrier()

        base0 = wid * EW

        @pl.loop(0, NCHUNK)
        def _chunk(j):
            base = base0 + j * CHUNK
            pltpu.async_copy(src_h.at[pl.ds(base, CHUNK)], srcb, sem).wait()
            pltpu.async_copy(dst_h.at[pl.ds(base, CHUNK)], dstb, sem).wait()
            pltpu.async_copy(x_h.at[srcb], rows, sem).wait()
            pltpu.async_copy(rows, accx.at[dstb], sem, add=True).wait()

        plsc.subcore_barrier()

        @pl.loop(0, ZITER)
        def _dump(k):
            pltpu.async_copy(accx.at[pl.ds(r0 + k * ZROWS, ZROWS)], rows,
                             sem).wait()
            pltpu.async_copy(rows, sx_out.at[m, pl.ds(o0 + k * ZROWS, ZROWS)],
                             sem).wait()

        plsc.subcore_barrier()

    _one_type(0, src0, dst0, xa)
    _one_type(1, src1, dst1, xp)
    _one_type(2, src2, dst2, xp)


_scx_scatter = pl.kernel(
    _scx_body,
    out_type=jax.ShapeDtypeStruct((3, NC * NPAD, D_IN), jnp.float32),
    mesh=plsc.VectorSubcoreMesh(core_axis_name="c", subcore_axis_name="s"),
    scratch_types=[
        pltpu.VMEM_SHARED((NPAD, D_IN), jnp.float32),
        pltpu.VMEM((CHUNK,), jnp.int32),
        pltpu.VMEM((CHUNK,), jnp.int32),
        pltpu.VMEM((CHUNK, D_IN), jnp.float32),
        pltpu.SemaphoreType.DMA,
    ],
    name="hetero_gnn_scx",
)


def _sce_body(dst0, ef0, dst1, ef1, dst2, ef2,
              se_out, acce, dstb, efb16, efb, sem):
    c = lax.axis_index("c")
    s = lax.axis_index("s")
    wid = c * NS + s
    zero16 = jnp.zeros((16,), jnp.float32)
    r0 = s * RPT
    o0 = c * NPAD + s * RPT

    def _one_type(m, dst_h, ef_h):
        def _fill(i, _):
            for j in range(D_IN // 16):
                efb[i, pl.ds(j * 16, 16)] = zero16
            return 0

        lax.fori_loop(0, ZROWS, _fill, 0)

        @pl.loop(0, ZITER)
        def _zero(k):
            pltpu.async_copy(efb, acce.at[pl.ds(r0 + k * ZROWS, ZROWS)],
                             sem).wait()

        plsc.subcore_barrier()

        base0 = wid * EW

        @pl.loop(0, NCHUNK)
        def _chunk(j):
            base = base0 + j * CHUNK
            pltpu.async_copy(dst_h.at[pl.ds(base, CHUNK)], dstb, sem).wait()
            pltpu.async_copy(ef_h.at[pl.ds(base, CHUNK)], efb16, sem).wait()

            def _widen(i, _):
                efb[i, pl.ds(0, 16)] = efb16[i, :]
                return 0

            lax.fori_loop(0, CHUNK, _widen, 0)
            pltpu.async_copy(efb, acce.at[dstb], sem, add=True).wait()

        plsc.subcore_barrier()

        @pl.loop(0, ZITER)
        def _dump(k):
            pltpu.async_copy(acce.at[pl.ds(r0 + k * ZROWS, ZROWS)], efb,
                             sem).wait()
            pltpu.async_copy(efb, se_out.at[m, pl.ds(o0 + k * ZROWS, ZROWS)],
                             sem).wait()

        plsc.subcore_barrier()

    _one_type(0, dst0, ef0)
    _one_type(1, dst1, ef1)
    _one_type(2, dst2, ef2)


_sce_scatter = pl.kernel(
    _sce_body,
    out_type=jax.ShapeDtypeStruct((3, NC * NPAD, D_IN), jnp.float32),
    mesh=plsc.VectorSubcoreMesh(core_axis_name="c", subcore_axis_name="s"),
    scratch_types=[
        pltpu.VMEM_SHARED((NPAD, D_IN), jnp.float32),
        pltpu.VMEM((CHUNK,), jnp.int32),
        pltpu.VMEM((CHUNK, D_EDGE), jnp.float32),
        pltpu.VMEM((CHUNK, D_IN), jnp.float32),
        pltpu.SemaphoreType.DMA,
    ],
    name="hetero_gnn_sce",
)


def _tc_body(sx, se, xa, xp, wn, we, ws, ba, bp, outa, outp):
    f32 = jnp.float32
    acc = jnp.dot(sx[0, 0] + sx[0, 1], wn[0], preferred_element_type=f32)
    acc += jnp.dot(sx[1, 0] + sx[1, 1], wn[1], preferred_element_type=f32)
    acc += jnp.dot(se[0, 0] + se[0, 1], we[0], preferred_element_type=f32)
    acc += jnp.dot(se[1, 0] + se[1, 1], we[1], preferred_element_type=f32)
    acc += jnp.dot(xp[...], ws[0] + ws[1], preferred_element_type=f32)
    outp[...] = acc + bp[...]
    acc = jnp.dot(sx[2, 0] + sx[2, 1], wn[2], preferred_element_type=f32)
    acc += jnp.dot(se[2, 0] + se[2, 1], we[2], preferred_element_type=f32)
    acc += jnp.dot(xa[...], ws[2], preferred_element_type=f32)
    outa[...] = acc + ba[...]


_BR = 1000


def _tc_combine(sx, se, xa, xp, wn, we, ws, ba, bp):
    nb = N_NODE // _BR
    return pl.pallas_call(
        _tc_body,
        grid=(nb,),
        in_specs=[
            pl.BlockSpec((3, NC, _BR, D_IN), lambda i: (0, 0, i, 0)),
            pl.BlockSpec((3, NC, _BR, D_EDGE), lambda i: (0, 0, i, 0)),
            pl.BlockSpec((_BR, D_IN), lambda i: (i, 0)),
            pl.BlockSpec((_BR, D_IN), lambda i: (i, 0)),
            pl.BlockSpec((3, D_IN, D_OUT), lambda i: (0, 0, 0)),
            pl.BlockSpec((3, D_EDGE, D_OUT), lambda i: (0, 0, 0)),
            pl.BlockSpec((3, D_IN, D_OUT), lambda i: (0, 0, 0)),
            pl.BlockSpec((1, D_OUT), lambda i: (0, 0)),
            pl.BlockSpec((1, D_OUT), lambda i: (0, 0)),
        ],
        out_specs=[
            pl.BlockSpec((_BR, D_OUT), lambda i: (i, 0)),
            pl.BlockSpec((_BR, D_OUT), lambda i: (i, 0)),
        ],
        out_shape=[
            jax.ShapeDtypeStruct((N_NODE, D_OUT), jnp.float32),
            jax.ShapeDtypeStruct((N_NODE, D_OUT), jnp.float32),
        ],
        name="hetero_gnn_tc_combine",
    )(sx, se, xa, xp, wn, we, ws, ba, bp)


def kernel(x_author, x_paper, edge_index_m0, edge_index_m1, edge_index_m2,
           edge_feat_m0, edge_feat_m1, edge_feat_m2,
           Wn0, Ws0, We0, b0, Wn1, Ws1, We1, b1, Wn2, Ws2, We2, b2):
    i32 = jnp.int32
    src0, dst0 = edge_index_m0[0].astype(i32), edge_index_m0[1].astype(i32)
    src1, dst1 = edge_index_m1[0].astype(i32), edge_index_m1[1].astype(i32)
    src2, dst2 = edge_index_m2[0].astype(i32), edge_index_m2[1].astype(i32)

    sx_p = _scx_scatter(x_author, x_paper, src0, dst0, src1, dst1, src2, dst2)
    se_p = _sce_scatter(dst0, edge_feat_m0, dst1, edge_feat_m1,
                        dst2, edge_feat_m2)
    sx = sx_p.reshape(3, NC, NPAD, D_IN)[:, :, :N_NODE]
    se = se_p.reshape(3, NC, NPAD, D_IN)[:, :, :N_NODE, :D_EDGE]

    wn = jnp.stack([Wn0, Wn1, Wn2])
    we = jnp.stack([We0, We1, We2])
    ws = jnp.stack([Ws0, Ws1, Ws2])
    ba = b2.reshape(1, D_OUT)
    bp = (b0 + b1).reshape(1, D_OUT)

    emb_author, emb_paper = _tc_combine(sx, se, x_author, x_paper,
                                        wn, we, ws, ba, bp)
    return (emb_author, emb_paper)


# speedup vs baseline: 3.8747x; 1.6324x over previous
import jax
import jax.numpy as jnp
from jax import lax
from jax.experimental import pallas as pl
from jax.experimental.pallas import tpu as pltpu, tpu_sc as plsc

N_NODE = 10000
D_IN = 128
D_OUT = 128
D_EDGE = 16
E = 320000

NC = 2
NS = 16
NW = NC * NS
EW = E // NW
CHUNK = 80
NCHUNK = EW // CHUNK
NPAD = 10240
RPT = NPAD // NS
ZROWS = 80
ZITER = RPT // ZROWS


NPAIR = (NCHUNK - 1) // 2


def _scx_body(xa, xp, src0, dst0, src1, dst1, src2, dst2,
              sx_out, accx, srcb0, dstb0, srcb1, dstb1,
              rows0, rows1, semg0, semg1, sem):
    c = lax.axis_index("c")
    s = lax.axis_index("s")
    wid = c * NS + s
    zero16 = jnp.zeros((16,), jnp.float32)
    r0 = s * RPT
    o0 = c * NPAD + s * RPT

    def _one_type(m, src_h, dst_h, x_h):
        def _fill(i, _):
            for j in range(D_IN // 16):
                rows0[i, pl.ds(j * 16, 16)] = zero16
            return 0

        lax.fori_loop(0, ZROWS, _fill, 0)

        @pl.loop(0, ZITER)
        def _zero(k):
            pltpu.async_copy(rows0, accx.at[pl.ds(r0 + k * ZROWS, ZROWS)],
                             sem).wait()

        plsc.subcore_barrier()

        base0 = wid * EW

        def _load_issue(j, srcb, dstb, rows, semg):
            base = base0 + j * CHUNK
            pltpu.async_copy(src_h.at[pl.ds(base, CHUNK)], srcb, sem).wait()
            pltpu.async_copy(dst_h.at[pl.ds(base, CHUNK)], dstb, sem).wait()
            pltpu.async_copy(x_h.at[srcb], rows, semg)

        def _finish(srcb, dstb, rows, semg):
            pltpu.make_async_copy(x_h.at[srcb], rows, semg).wait()
            pltpu.async_copy(rows, accx.at[dstb], sem, add=True).wait()

        _load_issue(0, srcb0, dstb0, rows0, semg0)

        @pl.loop(0, NPAIR)
        def _pair(p):
            _load_issue(2 * p + 1, srcb1, dstb1, rows1, semg1)
            _finish(srcb0, dstb0, rows0, semg0)
            _load_issue(2 * p + 2, srcb0, dstb0, rows0, semg0)
            _finish(srcb1, dstb1, rows1, semg1)

        _finish(srcb0, dstb0, rows0, semg0)
        plsc.subcore_barrier()

        @pl.loop(0, ZITER)
        def _dump(k):
            pltpu.async_copy(accx.at[pl.ds(r0 + k * ZROWS, ZROWS)], rows0,
                             sem).wait()
            pltpu.async_copy(rows0, sx_out.at[m, pl.ds(o0 + k * ZROWS, ZROWS)],
                             sem).wait()

        plsc.subcore_barrier()

    _one_type(0, src0, dst0, xa)
    _one_type(1, src1, dst1, xp)
    _one_type(2, src2, dst2, xp)


_scx_scatter = pl.kernel(
    _scx_body,
    out_type=jax.ShapeDtypeStruct((3, NC * NPAD, D_IN), jnp.float32),
    mesh=plsc.VectorSubcoreMesh(core_axis_name="c", subcore_axis_name="s"),
    scratch_types=[
        pltpu.VMEM_SHARED((NPAD, D_IN), jnp.float32),
        pltpu.VMEM((CHUNK,), jnp.int32),
        pltpu.VMEM((CHUNK,), jnp.int32),
        pltpu.VMEM((CHUNK,), jnp.int32),
        pltpu.VMEM((CHUNK,), jnp.int32),
        pltpu.VMEM((CHUNK, D_IN), jnp.float32),
        pltpu.VMEM((CHUNK, D_IN), jnp.float32),
        pltpu.SemaphoreType.DMA,
        pltpu.SemaphoreType.DMA,
        pltpu.SemaphoreType.DMA,
    ],
    name="hetero_gnn_scx",
)


def _sce_body(dst0, ef0, dst1, ef1, dst2, ef2,
              se_out, acce, dstb0, dstb1, efb16_0, efb16_1, efb0, efb1,
              seme0, seme1, sem):
    c = lax.axis_index("c")
    s = lax.axis_index("s")
    wid = c * NS + s
    zero16 = jnp.zeros((16,), jnp.float32)
    r0 = s * RPT
    o0 = c * NPAD + s * RPT

    def _one_type(m, dst_h, ef_h):
        def _fill(i, _):
            for j in range(D_IN // 16):
                efb0[i, pl.ds(j * 16, 16)] = zero16
                efb1[i, pl.ds(j * 16, 16)] = zero16
            return 0

        lax.fori_loop(0, ZROWS, _fill, 0)

        @pl.loop(0, ZITER)
        def _zero(k):
            pltpu.async_copy(efb0, acce.at[pl.ds(r0 + k * ZROWS, ZROWS)],
                             sem).wait()

        plsc.subcore_barrier()

        base0 = wid * EW

        def _load_issue(j, dstb, efb16, seme):
            base = base0 + j * CHUNK
            pltpu.async_copy(ef_h.at[pl.ds(base, CHUNK)], efb16, seme)
            pltpu.async_copy(dst_h.at[pl.ds(base, CHUNK)], dstb, sem).wait()

        def _finish(dstb, efb16, efb, seme):
            pltpu.make_async_copy(ef_h.at[pl.ds(0, CHUNK)], efb16, seme).wait()

            def _widen(i, _):
                efb[i, pl.ds(0, 16)] = efb16[i, :]
                return 0

            lax.fori_loop(0, CHUNK, _widen, 0)
            pltpu.async_copy(efb, acce.at[dstb], sem, add=True).wait()

        _load_issue(0, dstb0, efb16_0, seme0)

        @pl.loop(0, NPAIR)
        def _pair(p):
            _load_issue(2 * p + 1, dstb1, efb16_1, seme1)
            _finish(dstb0, efb16_0, efb0, seme0)
            _load_issue(2 * p + 2, dstb0, efb16_0, seme0)
            _finish(dstb1, efb16_1, efb1, seme1)

        _finish(dstb0, efb16_0, efb0, seme0)
        plsc.subcore_barrier()

        @pl.loop(0, ZITER)
        def _dump(k):
            pltpu.async_copy(acce.at[pl.ds(r0 + k * ZROWS, ZROWS)], efb0,
                             sem).wait()
            pltpu.async_copy(efb0, se_out.at[m, pl.ds(o0 + k * ZROWS, ZROWS)],
                             sem).wait()

        plsc.subcore_barrier()

    _one_type(0, dst0, ef0)
    _one_type(1, dst1, ef1)
    _one_type(2, dst2, ef2)


_sce_scatter = pl.kernel(
    _sce_body,
    out_type=jax.ShapeDtypeStruct((3, NC * NPAD, D_IN), jnp.float32),
    mesh=plsc.VectorSubcoreMesh(core_axis_name="c", subcore_axis_name="s"),
    scratch_types=[
        pltpu.VMEM_SHARED((NPAD, D_IN), jnp.float32),
        pltpu.VMEM((CHUNK,), jnp.int32),
        pltpu.VMEM((CHUNK,), jnp.int32),
        pltpu.VMEM((CHUNK, D_EDGE), jnp.float32),
        pltpu.VMEM((CHUNK, D_EDGE), jnp.float32),
        pltpu.VMEM((CHUNK, D_IN), jnp.float32),
        pltpu.VMEM((CHUNK, D_IN), jnp.float32),
        pltpu.SemaphoreType.DMA,
        pltpu.SemaphoreType.DMA,
        pltpu.SemaphoreType.DMA,
    ],
    name="hetero_gnn_sce",
)


def _tc_body(sx, se, xa, xp, wn, we, ws, ba, bp, outa, outp):
    f32 = jnp.float32
    acc = jnp.dot(sx[0, 0] + sx[0, 1], wn[0], preferred_element_type=f32)
    acc += jnp.dot(sx[1, 0] + sx[1, 1], wn[1], preferred_element_type=f32)
    acc += jnp.dot(se[0, 0] + se[0, 1], we[0], preferred_element_type=f32)
    acc += jnp.dot(se[1, 0] + se[1, 1], we[1], preferred_element_type=f32)
    acc += jnp.dot(xp[...], ws[0] + ws[1], preferred_element_type=f32)
    outp[...] = acc + bp[...]
    acc = jnp.dot(sx[2, 0] + sx[2, 1], wn[2], preferred_element_type=f32)
    acc += jnp.dot(se[2, 0] + se[2, 1], we[2], preferred_element_type=f32)
    acc += jnp.dot(xa[...], ws[2], preferred_element_type=f32)
    outa[...] = acc + ba[...]


_BR = 1000


def _tc_combine(sx, se, xa, xp, wn, we, ws, ba, bp):
    nb = N_NODE // _BR
    return pl.pallas_call(
        _tc_body,
        grid=(nb,),
        in_specs=[
            pl.BlockSpec((3, NC, _BR, D_IN), lambda i: (0, 0, i, 0)),
            pl.BlockSpec((3, NC, _BR, D_EDGE), lambda i: (0, 0, i, 0)),
            pl.BlockSpec((_BR, D_IN), lambda i: (i, 0)),
            pl.BlockSpec((_BR, D_IN), lambda i: (i, 0)),
            pl.BlockSpec((3, D_IN, D_OUT), lambda i: (0, 0, 0)),
            pl.BlockSpec((3, D_EDGE, D_OUT), lambda i: (0, 0, 0)),
            pl.BlockSpec((3, D_IN, D_OUT), lambda i: (0, 0, 0)),
            pl.BlockSpec((1, D_OUT), lambda i: (0, 0)),
            pl.BlockSpec((1, D_OUT), lambda i: (0, 0)),
        ],
        out_specs=[
            pl.BlockSpec((_BR, D_OUT), lambda i: (i, 0)),
            pl.BlockSpec((_BR, D_OUT), lambda i: (i, 0)),
        ],
        out_shape=[
            jax.ShapeDtypeStruct((N_NODE, D_OUT), jnp.float32),
            jax.ShapeDtypeStruct((N_NODE, D_OUT), jnp.float32),
        ],
        name="hetero_gnn_tc_combine",
    )(sx, se, xa, xp, wn, we, ws, ba, bp)


def kernel(x_author, x_paper, edge_index_m0, edge_index_m1, edge_index_m2,
           edge_feat_m0, edge_feat_m1, edge_feat_m2,
           Wn0, Ws0, We0, b0, Wn1, Ws1, We1, b1, Wn2, Ws2, We2, b2):
    i32 = jnp.int32
    src0, dst0 = edge_index_m0[0].astype(i32), edge_index_m0[1].astype(i32)
    src1, dst1 = edge_index_m1[0].astype(i32), edge_index_m1[1].astype(i32)
    src2, dst2 = edge_index_m2[0].astype(i32), edge_index_m2[1].astype(i32)

    sx_p = _scx_scatter(x_author, x_paper, src0, dst0, src1, dst1, src2, dst2)
    se_p = _sce_scatter(dst0, edge_feat_m0, dst1, edge_feat_m1,
                        dst2, edge_feat_m2)
    sx = sx_p.reshape(3, NC, NPAD, D_IN)[:, :, :N_NODE]
    se = se_p.reshape(3, NC, NPAD, D_IN)[:, :, :N_NODE, :D_EDGE]

    wn = jnp.stack([Wn0, Wn1, Wn2])
    we = jnp.stack([We0, We1, We2])
    ws = jnp.stack([Ws0, Ws1, Ws2])
    ba = b2.reshape(1, D_OUT)
    bp = (b0 + b1).reshape(1, D_OUT)

    emb_author, emb_paper = _tc_combine(sx, se, x_author, x_paper,
                                        wn, we, ws, ba, bp)
    return (emb_author, emb_paper)


# speedup vs baseline: 4.8282x; 1.2461x over previous
import jax
import jax.numpy as jnp
from jax import lax
from jax.experimental import pallas as pl
from jax.experimental.pallas import tpu as pltpu, tpu_sc as plsc

N_NODE = 10000
D_IN = 128
D_OUT = 128
D_EDGE = 16
E = 320000

NC = 2
NS = 16
NW = NC * NS
EW = E // NW
CHUNK = 80
NCHUNK = EW // CHUNK
NPAD = 10240
RPT = NPAD // NS
ZROWS = 80
ZITER = RPT // ZROWS


NPAIR = (NCHUNK - 1) // 2


def _scx_body(xa, xp, src0, dst0, src1, dst1, src2, dst2,
              sx_out, accx, srcb0, dstb0, srcb1, dstb1,
              rows0, rows1, semg0, semg1, semi0, semi1, sem):
    c = lax.axis_index("c")
    s = lax.axis_index("s")
    wid = c * NS + s
    zero16 = jnp.zeros((16,), jnp.float32)
    r0 = s * RPT
    o0 = c * NPAD + s * RPT

    def _one_type(m, src_h, dst_h, x_h):
        def _fill(i, _):
            for j in range(D_IN // 16):
                rows0[i, pl.ds(j * 16, 16)] = zero16
            return 0

        lax.fori_loop(0, ZROWS, _fill, 0)

        @pl.loop(0, ZITER)
        def _zero(k):
            pltpu.async_copy(rows0, accx.at[pl.ds(r0 + k * ZROWS, ZROWS)],
                             sem).wait()

        plsc.subcore_barrier()

        base0 = wid * EW

        def _idx_load(j, srcb, dstb, semi):
            base = jnp.minimum(base0 + j * CHUNK, E - CHUNK)
            pltpu.async_copy(src_h.at[pl.ds(base, CHUNK)], srcb, semi)
            pltpu.async_copy(dst_h.at[pl.ds(base, CHUNK)], dstb, semi)

        def _wait_idx(srcb, dstb, semi):
            pltpu.make_async_copy(src_h.at[pl.ds(0, CHUNK)], srcb, semi).wait()
            pltpu.make_async_copy(dst_h.at[pl.ds(0, CHUNK)], dstb, semi).wait()

        def _scatter(dstb, rows, semg):
            pltpu.make_async_copy(x_h.at[srcb0], rows, semg).wait()
            pltpu.async_copy(rows, accx.at[dstb], sem, add=True).wait()

        _idx_load(0, srcb0, dstb0, semi0)
        _wait_idx(srcb0, dstb0, semi0)
        pltpu.async_copy(x_h.at[srcb0], rows0, semg0)
        _idx_load(1, srcb1, dstb1, semi1)

        @pl.loop(0, NPAIR)
        def _pair(p):
            _wait_idx(srcb1, dstb1, semi1)
            pltpu.async_copy(x_h.at[srcb1], rows1, semg1)
            _scatter(dstb0, rows0, semg0)
            _idx_load(2 * p + 2, srcb0, dstb0, semi0)
            _scatter(dstb1, rows1, semg1)
            _wait_idx(srcb0, dstb0, semi0)
            pltpu.async_copy(x_h.at[srcb0], rows0, semg0)
            _idx_load(2 * p + 3, srcb1, dstb1, semi1)

        _scatter(dstb0, rows0, semg0)
        _wait_idx(srcb1, dstb1, semi1)
        plsc.subcore_barrier()

        @pl.loop(0, ZITER)
        def _dump(k):
            pltpu.async_copy(accx.at[pl.ds(r0 + k * ZROWS, ZROWS)], rows0,
                             sem).wait()
            pltpu.async_copy(rows0, sx_out.at[m, pl.ds(o0 + k * ZROWS, ZROWS)],
                             sem).wait()

        plsc.subcore_barrier()

    _one_type(0, src0, dst0, xa)
    _one_type(1, src1, dst1, xp)
    _one_type(2, src2, dst2, xp)


_scx_scatter = pl.kernel(
    _scx_body,
    out_type=jax.ShapeDtypeStruct((3, NC * NPAD, D_IN), jnp.float32),
    mesh=plsc.VectorSubcoreMesh(core_axis_name="c", subcore_axis_name="s"),
    scratch_types=[
        pltpu.VMEM_SHARED((NPAD, D_IN), jnp.float32),
        pltpu.VMEM((CHUNK,), jnp.int32),
        pltpu.VMEM((CHUNK,), jnp.int32),
        pltpu.VMEM((CHUNK,), jnp.int32),
        pltpu.VMEM((CHUNK,), jnp.int32),
        pltpu.VMEM((CHUNK, D_IN), jnp.float32),
        pltpu.VMEM((CHUNK, D_IN), jnp.float32),
        pltpu.SemaphoreType.DMA,
        pltpu.SemaphoreType.DMA,
        pltpu.SemaphoreType.DMA,
        pltpu.SemaphoreType.DMA,
        pltpu.SemaphoreType.DMA,
    ],
    name="hetero_gnn_scx",
)


def _sce_body(dst0, ef0, dst1, ef1, dst2, ef2,
              se_out, acce, dstb0, dstb1, efb16_0, efb16_1, efb0, efb1,
              seme0, seme1, sem):
    c = lax.axis_index("c")
    s = lax.axis_index("s")
    wid = c * NS + s
    zero16 = jnp.zeros((16,), jnp.float32)
    r0 = s * RPT
    o0 = c * NPAD + s * RPT

    def _one_type(m, dst_h, ef_h):
        def _fill(i, _):
            for j in range(D_IN // 16):
                efb0[i, pl.ds(j * 16, 16)] = zero16
                efb1[i, pl.ds(j * 16, 16)] = zero16
            return 0

        lax.fori_loop(0, ZROWS, _fill, 0)

        @pl.loop(0, ZITER)
        def _zero(k):
            pltpu.async_copy(efb0, acce.at[pl.ds(r0 + k * ZROWS, ZROWS)],
                             sem).wait()

        plsc.subcore_barrier()

        base0 = wid * EW

        def _load(j, dstb, efb16, seme):
            base = jnp.minimum(base0 + j * CHUNK, E - CHUNK)
            pltpu.async_copy(ef_h.at[pl.ds(base, CHUNK)], efb16, seme)
            pltpu.async_copy(dst_h.at[pl.ds(base, CHUNK)], dstb, seme)

        def _wait_load(dstb, efb16, seme):
            pltpu.make_async_copy(ef_h.at[pl.ds(0, CHUNK)], efb16, seme).wait()
            pltpu.make_async_copy(dst_h.at[pl.ds(0, CHUNK)], dstb, seme).wait()

        def _finish(dstb, efb16, efb, seme):
            _wait_load(dstb, efb16, seme)

            def _widen(i, _):
                efb[i, pl.ds(0, 16)] = efb16[i, :]
                return 0

            lax.fori_loop(0, CHUNK, _widen, 0, unroll=8)
            pltpu.async_copy(efb, acce.at[dstb], sem, add=True).wait()

        _load(0, dstb0, efb16_0, seme0)
        _load(1, dstb1, efb16_1, seme1)

        @pl.loop(0, NPAIR)
        def _pair(p):
            _finish(dstb0, efb16_0, efb0, seme0)
            _load(2 * p + 2, dstb0, efb16_0, seme0)
            _finish(dstb1, efb16_1, efb1, seme1)
            _load(2 * p + 3, dstb1, efb16_1, seme1)

        _finish(dstb0, efb16_0, efb0, seme0)
        _wait_load(dstb1, efb16_1, seme1)
        plsc.subcore_barrier()

        @pl.loop(0, ZITER)
        def _dump(k):
            pltpu.async_copy(acce.at[pl.ds(r0 + k * ZROWS, ZROWS)], efb0,
                             sem).wait()
            pltpu.async_copy(efb0, se_out.at[m, pl.ds(o0 + k * ZROWS, ZROWS)],
                             sem).wait()

        plsc.subcore_barrier()

    _one_type(0, dst0, ef0)
    _one_type(1, dst1, ef1)
    _one_type(2, dst2, ef2)


_sce_scatter = pl.kernel(
    _sce_body,
    out_type=jax.ShapeDtypeStruct((3, NC * NPAD, D_IN), jnp.float32),
    mesh=plsc.VectorSubcoreMesh(core_axis_name="c", subcore_axis_name="s"),
    scratch_types=[
        pltpu.VMEM_SHARED((NPAD, D_IN), jnp.float32),
        pltpu.VMEM((CHUNK,), jnp.int32),
        pltpu.VMEM((CHUNK,), jnp.int32),
        pltpu.VMEM((CHUNK, D_EDGE), jnp.float32),
        pltpu.VMEM((CHUNK, D_EDGE), jnp.float32),
        pltpu.VMEM((CHUNK, D_IN), jnp.float32),
        pltpu.VMEM((CHUNK, D_IN), jnp.float32),
        pltpu.SemaphoreType.DMA,
        pltpu.SemaphoreType.DMA,
        pltpu.SemaphoreType.DMA,
    ],
    name="hetero_gnn_sce",
)


def _tc_body(sx, se, xa, xp, wn, we, ws, ba, bp, outa, outp):
    f32 = jnp.float32
    acc = jnp.dot(sx[0, 0] + sx[0, 1], wn[0], preferred_element_type=f32)
    acc += jnp.dot(sx[1, 0] + sx[1, 1], wn[1], preferred_element_type=f32)
    acc += jnp.dot(se[0, 0] + se[0, 1], we[0], preferred_element_type=f32)
    acc += jnp.dot(se[1, 0] + se[1, 1], we[1], preferred_element_type=f32)
    acc += jnp.dot(xp[...], ws[0] + ws[1], preferred_element_type=f32)
    outp[...] = acc + bp[...]
    acc = jnp.dot(sx[2, 0] + sx[2, 1], wn[2], preferred_element_type=f32)
    acc += jnp.dot(se[2, 0] + se[2, 1], we[2], preferred_element_type=f32)
    acc += jnp.dot(xa[...], ws[2], preferred_element_type=f32)
    outa[...] = acc + ba[...]


_BR = 1000


def _tc_combine(sx, se, xa, xp, wn, we, ws, ba, bp):
    nb = N_NODE // _BR
    return pl.pallas_call(
        _tc_body,
        grid=(nb,),
        in_specs=[
            pl.BlockSpec((3, NC, _BR, D_IN), lambda i: (0, 0, i, 0)),
            pl.BlockSpec((3, NC, _BR, D_EDGE), lambda i: (0, 0, i, 0)),
            pl.BlockSpec((_BR, D_IN), lambda i: (i, 0)),
            pl.BlockSpec((_BR, D_IN), lambda i: (i, 0)),
            pl.BlockSpec((3, D_IN, D_OUT), lambda i: (0, 0, 0)),
            pl.BlockSpec((3, D_EDGE, D_OUT), lambda i: (0, 0, 0)),
            pl.BlockSpec((3, D_IN, D_OUT), lambda i: (0, 0, 0)),
            pl.BlockSpec((1, D_OUT), lambda i: (0, 0)),
            pl.BlockSpec((1, D_OUT), lambda i: (0, 0)),
        ],
        out_specs=[
            pl.BlockSpec((_BR, D_OUT), lambda i: (i, 0)),
            pl.BlockSpec((_BR, D_OUT), lambda i: (i, 0)),
        ],
        out_shape=[
            jax.ShapeDtypeStruct((N_NODE, D_OUT), jnp.float32),
            jax.ShapeDtypeStruct((N_NODE, D_OUT), jnp.float32),
        ],
        name="hetero_gnn_tc_combine",
    )(sx, se, xa, xp, wn, we, ws, ba, bp)


def kernel(x_author, x_paper, edge_index_m0, edge_index_m1, edge_index_m2,
           edge_feat_m0, edge_feat_m1, edge_feat_m2,
           Wn0, Ws0, We0, b0, Wn1, Ws1, We1, b1, Wn2, Ws2, We2, b2):
    i32 = jnp.int32
    src0, dst0 = edge_index_m0[0].astype(i32), edge_index_m0[1].astype(i32)
    src1, dst1 = edge_index_m1[0].astype(i32), edge_index_m1[1].astype(i32)
    src2, dst2 = edge_index_m2[0].astype(i32), edge_index_m2[1].astype(i32)

    sx_p = _scx_scatter(x_author, x_paper, src0, dst0, src1, dst1, src2, dst2)
    se_p = _sce_scatter(dst0, edge_feat_m0, dst1, edge_feat_m1,
                        dst2, edge_feat_m2)
    sx = sx_p.reshape(3, NC, NPAD, D_IN)[:, :, :N_NODE]
    se = se_p.reshape(3, NC, NPAD, D_IN)[:, :, :N_NODE, :D_EDGE]

    wn = jnp.stack([Wn0, Wn1, Wn2])
    we = jnp.stack([We0, We1, We2])
    ws = jnp.stack([Ws0, Ws1, Ws2])
    ba = b2.reshape(1, D_OUT)
    bp = (b0 + b1).reshape(1, D_OUT)

    emb_author, emb_paper = _tc_combine(sx, se, x_author, x_paper,
                                        wn, we, ws, ba, bp)
    return (emb_author, emb_paper)


# speedup vs baseline: 4.9662x; 1.0286x over previous
import jax
import jax.numpy as jnp
from jax import lax
from jax.experimental import pallas as pl
from jax.experimental.pallas import tpu as pltpu, tpu_sc as plsc

N_NODE = 10000
D_IN = 128
D_OUT = 128
D_EDGE = 16
E = 320000

NC = 2
NS = 16
NW = NC * NS
EW = E // NW
CHUNK = 80
NCHUNK = EW // CHUNK
NPAD = 10240
RPT = NPAD // NS
ZROWS = 80
ZITER = RPT // ZROWS


NPAIR = (NCHUNK - 1) // 2


def _scx_body(xa, xp, src0, dst0, src1, dst1, src2, dst2,
              sx_out, accx, srcb0, dstb0, srcb1, dstb1,
              rows0, rows1, semg0, semg1, semi0, semi1, sem):
    c = lax.axis_index("c")
    s = lax.axis_index("s")
    wid = c * NS + s
    zero16 = jnp.zeros((16,), jnp.float32)
    r0 = s * RPT
    o0 = c * NPAD + s * RPT

    def _one_type(m, src_h, dst_h, x_h):
        def _fill(i, _):
            for j in range(D_IN // 16):
                rows0[i, pl.ds(j * 16, 16)] = zero16
            return 0

        lax.fori_loop(0, ZROWS, _fill, 0)

        @pl.loop(0, ZITER)
        def _zero(k):
            pltpu.async_copy(rows0, accx.at[pl.ds(r0 + k * ZROWS, ZROWS)],
                             sem).wait()

        plsc.subcore_barrier()

        base0 = wid * EW

        def _idx_load(j, srcb, dstb, semi):
            base = jnp.minimum(base0 + j * CHUNK, E - CHUNK)
            pltpu.async_copy(src_h.at[pl.ds(base, CHUNK)], srcb, semi)
            pltpu.async_copy(dst_h.at[pl.ds(base, CHUNK)], dstb, semi)

        def _wait_idx(srcb, dstb, semi):
            pltpu.make_async_copy(src_h.at[pl.ds(0, CHUNK)], srcb, semi).wait()
            pltpu.make_async_copy(dst_h.at[pl.ds(0, CHUNK)], dstb, semi).wait()

        def _scatter(dstb, rows, semg):
            pltpu.make_async_copy(x_h.at[srcb0], rows, semg).wait()
            pltpu.async_copy(rows, accx.at[dstb], sem, add=True).wait()

        _idx_load(0, srcb0, dstb0, semi0)
        _wait_idx(srcb0, dstb0, semi0)
        pltpu.async_copy(x_h.at[srcb0], rows0, semg0)
        _idx_load(1, srcb1, dstb1, semi1)

        @pl.loop(0, NPAIR)
        def _pair(p):
            _wait_idx(srcb1, dstb1, semi1)
            pltpu.async_copy(x_h.at[srcb1], rows1, semg1)
            _scatter(dstb0, rows0, semg0)
            _idx_load(2 * p + 2, srcb0, dstb0, semi0)
            _scatter(dstb1, rows1, semg1)
            _wait_idx(srcb0, dstb0, semi0)
            pltpu.async_copy(x_h.at[srcb0], rows0, semg0)
            _idx_load(2 * p + 3, srcb1, dstb1, semi1)

        _scatter(dstb0, rows0, semg0)
        _wait_idx(srcb1, dstb1, semi1)
        plsc.subcore_barrier()

        @pl.loop(0, ZITER)
        def _dump(k):
            pltpu.async_copy(accx.at[pl.ds(r0 + k * ZROWS, ZROWS)], rows0,
                             sem).wait()
            pltpu.async_copy(rows0, sx_out.at[m, pl.ds(o0 + k * ZROWS, ZROWS)],
                             sem).wait()

        plsc.subcore_barrier()

    _one_type(0, src0, dst0, xa)
    _one_type(1, src1, dst1, xp)
    _one_type(2, src2, dst2, xp)


_scx_scatter = pl.kernel(
    _scx_body,
    out_type=jax.ShapeDtypeStruct((3, NC * NPAD, D_IN), jnp.float32),
    mesh=plsc.VectorSubcoreMesh(core_axis_name="c", subcore_axis_name="s"),
    scratch_types=[
        pltpu.VMEM_SHARED((NPAD, D_IN), jnp.float32),
        pltpu.VMEM((CHUNK,), jnp.int32),
        pltpu.VMEM((CHUNK,), jnp.int32),
        pltpu.VMEM((CHUNK,), jnp.int32),
        pltpu.VMEM((CHUNK,), jnp.int32),
        pltpu.VMEM((CHUNK, D_IN), jnp.float32),
        pltpu.VMEM((CHUNK, D_IN), jnp.float32),
        pltpu.SemaphoreType.DMA,
        pltpu.SemaphoreType.DMA,
        pltpu.SemaphoreType.DMA,
        pltpu.SemaphoreType.DMA,
        pltpu.SemaphoreType.DMA,
    ],
    name="hetero_gnn_scx",
)


def _sce_body(dst0, ef0, dst1, ef1, dst2, ef2,
              se_out, acce, dstb0, dstb1, efb16_0, efb16_1, efb0, efb1,
              seme0, seme1, sem):
    c = lax.axis_index("c")
    s = lax.axis_index("s")
    wid = c * NS + s
    zero16 = jnp.zeros((16,), jnp.float32)
    r0 = s * RPT
    o0 = c * NPAD + s * RPT

    def _one_type(m, dst_h, ef_h):
        def _fill(i, _):
            for j in range(D_IN // 16):
                efb0[i, pl.ds(j * 16, 16)] = zero16
                efb1[i, pl.ds(j * 16, 16)] = zero16
            return 0

        lax.fori_loop(0, ZROWS, _fill, 0)

        @pl.loop(0, ZITER)
        def _zero(k):
            pltpu.async_copy(efb0, acce.at[pl.ds(r0 + k * ZROWS, ZROWS)],
                             sem).wait()

        plsc.subcore_barrier()

        base0 = wid * EW

        def _load(j, dstb, efb16, seme):
            base = jnp.minimum(base0 + j * CHUNK, E - CHUNK)
            pltpu.async_copy(ef_h.at[pl.ds(base, CHUNK)], efb16, seme)
            pltpu.async_copy(dst_h.at[pl.ds(base, CHUNK)], dstb, seme)

        def _wait_load(dstb, efb16, seme):
            pltpu.make_async_copy(ef_h.at[pl.ds(0, CHUNK)], efb16, seme).wait()
            pltpu.make_async_copy(dst_h.at[pl.ds(0, CHUNK)], dstb, seme).wait()

        def _finish(dstb, efb16, efb, seme):
            _wait_load(dstb, efb16, seme)

            def _widen(i, _):
                efb[i, pl.ds(0, 16)] = efb16[i, :]
                return 0

            lax.fori_loop(0, CHUNK, _widen, 0, unroll=8)
            pltpu.async_copy(efb, acce.at[dstb], sem, add=True).wait()

        _load(0, dstb0, efb16_0, seme0)
        _load(1, dstb1, efb16_1, seme1)

        @pl.loop(0, NPAIR)
        def _pair(p):
            _finish(dstb0, efb16_0, efb0, seme0)
            _load(2 * p + 2, dstb0, efb16_0, seme0)
            _finish(dstb1, efb16_1, efb1, seme1)
            _load(2 * p + 3, dstb1, efb16_1, seme1)

        _finish(dstb0, efb16_0, efb0, seme0)
        _wait_load(dstb1, efb16_1, seme1)
        plsc.subcore_barrier()

        @pl.loop(0, ZITER)
        def _dump(k):
            pltpu.async_copy(acce.at[pl.ds(r0 + k * ZROWS, ZROWS)], efb0,
                             sem).wait()
            pltpu.async_copy(efb0, se_out.at[m, pl.ds(o0 + k * ZROWS, ZROWS)],
                             sem).wait()

        plsc.subcore_barrier()

    _one_type(0, dst0, ef0)
    _one_type(1, dst1, ef1)
    _one_type(2, dst2, ef2)


_sce_scatter = pl.kernel(
    _sce_body,
    out_type=jax.ShapeDtypeStruct((3, NC * NPAD, D_IN), jnp.float32),
    mesh=plsc.VectorSubcoreMesh(core_axis_name="c", subcore_axis_name="s"),
    scratch_types=[
        pltpu.VMEM_SHARED((NPAD, D_IN), jnp.float32),
        pltpu.VMEM((CHUNK,), jnp.int32),
        pltpu.VMEM((CHUNK,), jnp.int32),
        pltpu.VMEM((CHUNK, D_EDGE), jnp.float32),
        pltpu.VMEM((CHUNK, D_EDGE), jnp.float32),
        pltpu.VMEM((CHUNK, D_IN), jnp.float32),
        pltpu.VMEM((CHUNK, D_IN), jnp.float32),
        pltpu.SemaphoreType.DMA,
        pltpu.SemaphoreType.DMA,
        pltpu.SemaphoreType.DMA,
    ],
    name="hetero_gnn_sce",
)


def _tc_body(sx, se, xa, xp, wn, we, ws, ba, bp, outa, outp):
    f32 = jnp.float32
    acc = jnp.dot(sx[0, 0] + sx[0, 1], wn[0], preferred_element_type=f32)
    acc += jnp.dot(sx[1, 0] + sx[1, 1], wn[1], preferred_element_type=f32)
    acc += jnp.dot(se[0, 0, :, :D_EDGE] + se[0, 1, :, :D_EDGE], we[0],
                   preferred_element_type=f32)
    acc += jnp.dot(se[1, 0, :, :D_EDGE] + se[1, 1, :, :D_EDGE], we[1],
                   preferred_element_type=f32)
    acc += jnp.dot(xp[...], ws[0] + ws[1], preferred_element_type=f32)
    outp[...] = acc + bp[...]
    acc = jnp.dot(sx[2, 0] + sx[2, 1], wn[2], preferred_element_type=f32)
    acc += jnp.dot(se[2, 0, :, :D_EDGE] + se[2, 1, :, :D_EDGE], we[2],
                   preferred_element_type=f32)
    acc += jnp.dot(xa[...], ws[2], preferred_element_type=f32)
    outa[...] = acc + ba[...]


_BR = 1000


def _tc_combine(sx, se, xa, xp, wn, we, ws, ba, bp):
    nb = N_NODE // _BR
    return pl.pallas_call(
        _tc_body,
        grid=(nb,),
        in_specs=[
            pl.BlockSpec((3, NC, _BR, D_IN), lambda i: (0, 0, i, 0)),
            pl.BlockSpec((3, NC, _BR, D_IN), lambda i: (0, 0, i, 0)),
            pl.BlockSpec((_BR, D_IN), lambda i: (i, 0)),
            pl.BlockSpec((_BR, D_IN), lambda i: (i, 0)),
            pl.BlockSpec((3, D_IN, D_OUT), lambda i: (0, 0, 0)),
            pl.BlockSpec((3, D_EDGE, D_OUT), lambda i: (0, 0, 0)),
            pl.BlockSpec((3, D_IN, D_OUT), lambda i: (0, 0, 0)),
            pl.BlockSpec((1, D_OUT), lambda i: (0, 0)),
            pl.BlockSpec((1, D_OUT), lambda i: (0, 0)),
        ],
        out_specs=[
            pl.BlockSpec((_BR, D_OUT), lambda i: (i, 0)),
            pl.BlockSpec((_BR, D_OUT), lambda i: (i, 0)),
        ],
        out_shape=[
            jax.ShapeDtypeStruct((N_NODE, D_OUT), jnp.float32),
            jax.ShapeDtypeStruct((N_NODE, D_OUT), jnp.float32),
        ],
        name="hetero_gnn_tc_combine",
    )(sx, se, xa, xp, wn, we, ws, ba, bp)


def kernel(x_author, x_paper, edge_index_m0, edge_index_m1, edge_index_m2,
           edge_feat_m0, edge_feat_m1, edge_feat_m2,
           Wn0, Ws0, We0, b0, Wn1, Ws1, We1, b1, Wn2, Ws2, We2, b2):
    i32 = jnp.int32
    src0, dst0 = edge_index_m0[0].astype(i32), edge_index_m0[1].astype(i32)
    src1, dst1 = edge_index_m1[0].astype(i32), edge_index_m1[1].astype(i32)
    src2, dst2 = edge_index_m2[0].astype(i32), edge_index_m2[1].astype(i32)

    sx_p = _scx_scatter(x_author, x_paper, src0, dst0, src1, dst1, src2, dst2)
    se_p = _sce_scatter(dst0, edge_feat_m0, dst1, edge_feat_m1,
                        dst2, edge_feat_m2)
    sx = sx_p.reshape(3, NC, NPAD, D_IN)
    se = se_p.reshape(3, NC, NPAD, D_IN)

    wn = jnp.stack([Wn0, Wn1, Wn2])
    we = jnp.stack([We0, We1, We2])
    ws = jnp.stack([Ws0, Ws1, Ws2])
    ba = b2.reshape(1, D_OUT)
    bp = (b0 + b1).reshape(1, D_OUT)

    emb_author, emb_paper = _tc_combine(sx, se, x_author, x_paper,
                                        wn, we, ws, ba, bp)
    return (emb_author, emb_paper)


# speedup vs baseline: 4.9935x; 1.0055x over previous
import jax
import jax.numpy as jnp
from jax import lax
from jax.experimental import pallas as pl
from jax.experimental.pallas import tpu as pltpu, tpu_sc as plsc

N_NODE = 10000
D_IN = 128
D_OUT = 128
D_EDGE = 16
E = 320000

NC = 2
NS = 16
NW = NC * NS
EW = E // NW
CHUNK = 80
NCHUNK = EW // CHUNK
NPAD = 10240
RPT = NPAD // NS
ZROWS = 80
ZITER = RPT // ZROWS


NPAIR = (NCHUNK - 1) // 2


def _scx_body(xa, xp, src0, dst0, src1, dst1, src2, dst2,
              sx_out, accx, srcb0, dstb0, srcb1, dstb1,
              rows0, rows1, semg0, semg1, semi0, semi1, sem):
    c = lax.axis_index("c")
    s = lax.axis_index("s")
    wid = c * NS + s
    zero16 = jnp.zeros((16,), jnp.float32)
    r0 = s * RPT
    o0 = c * NPAD + s * RPT

    def _one_type(m, src_h, dst_h, x_h):
        def _fill(i, _):
            for j in range(D_IN // 16):
                rows0[i, pl.ds(j * 16, 16)] = zero16
            return 0

        lax.fori_loop(0, ZROWS, _fill, 0)

        @pl.loop(0, ZITER)
        def _zero(k):
            pltpu.async_copy(rows0, accx.at[pl.ds(r0 + k * ZROWS, ZROWS)],
                             sem)

        @pl.loop(0, ZITER)
        def _zero_drain(k):
            pltpu.make_async_copy(
                rows0, accx.at[pl.ds(r0, ZROWS)], sem).wait()

        plsc.subcore_barrier()

        base0 = wid * EW

        def _idx_load(j, srcb, dstb, semi):
            base = jnp.minimum(base0 + j * CHUNK, E - CHUNK)
            pltpu.async_copy(src_h.at[pl.ds(base, CHUNK)], srcb, semi)
            pltpu.async_copy(dst_h.at[pl.ds(base, CHUNK)], dstb, semi)

        def _wait_idx(srcb, dstb, semi):
            pltpu.make_async_copy(src_h.at[pl.ds(0, CHUNK)], srcb, semi).wait()
            pltpu.make_async_copy(dst_h.at[pl.ds(0, CHUNK)], dstb, semi).wait()

        def _scatter(dstb, rows, semg):
            pltpu.make_async_copy(x_h.at[srcb0], rows, semg).wait()
            pltpu.async_copy(rows, accx.at[dstb], sem, add=True).wait()

        _idx_load(0, srcb0, dstb0, semi0)
        _wait_idx(srcb0, dstb0, semi0)
        pltpu.async_copy(x_h.at[srcb0], rows0, semg0)
        _idx_load(1, srcb1, dstb1, semi1)

        @pl.loop(0, NPAIR)
        def _pair(p):
            _wait_idx(srcb1, dstb1, semi1)
            pltpu.async_copy(x_h.at[srcb1], rows1, semg1)
            _scatter(dstb0, rows0, semg0)
            _idx_load(2 * p + 2, srcb0, dstb0, semi0)
            _scatter(dstb1, rows1, semg1)
            _wait_idx(srcb0, dstb0, semi0)
            pltpu.async_copy(x_h.at[srcb0], rows0, semg0)
            _idx_load(2 * p + 3, srcb1, dstb1, semi1)

        _scatter(dstb0, rows0, semg0)
        _wait_idx(srcb1, dstb1, semi1)
        plsc.subcore_barrier()

        @pl.loop(0, ZITER // 2)
        def _dump(q):
            ra, rb = r0 + 2 * q * ZROWS, r0 + (2 * q + 1) * ZROWS
            oa, ob = o0 + 2 * q * ZROWS, o0 + (2 * q + 1) * ZROWS
            pltpu.async_copy(accx.at[pl.ds(ra, ZROWS)], rows0, semg0)
            pltpu.async_copy(accx.at[pl.ds(rb, ZROWS)], rows1, semg1)
            pltpu.make_async_copy(
                accx.at[pl.ds(ra, ZROWS)], rows0, semg0).wait()
            pltpu.async_copy(rows0, sx_out.at[m, pl.ds(oa, ZROWS)], semi0)
            pltpu.make_async_copy(
                accx.at[pl.ds(rb, ZROWS)], rows1, semg1).wait()
            pltpu.async_copy(rows1, sx_out.at[m, pl.ds(ob, ZROWS)], semi1)
            pltpu.make_async_copy(
                rows0, sx_out.at[m, pl.ds(oa, ZROWS)], semi0).wait()
            pltpu.make_async_copy(
                rows1, sx_out.at[m, pl.ds(ob, ZROWS)], semi1).wait()

        plsc.subcore_barrier()

    _one_type(0, src0, dst0, xa)
    _one_type(1, src1, dst1, xp)
    _one_type(2, src2, dst2, xp)


_scx_scatter = pl.kernel(
    _scx_body,
    out_type=jax.ShapeDtypeStruct((3, NC * NPAD, D_IN), jnp.float32),
    mesh=plsc.VectorSubcoreMesh(core_axis_name="c", subcore_axis_name="s"),
    scratch_types=[
        pltpu.VMEM_SHARED((NPAD, D_IN), jnp.float32),
        pltpu.VMEM((CHUNK,), jnp.int32),
        pltpu.VMEM((CHUNK,), jnp.int32),
        pltpu.VMEM((CHUNK,), jnp.int32),
        pltpu.VMEM((CHUNK,), jnp.int32),
        pltpu.VMEM((CHUNK, D_IN), jnp.float32),
        pltpu.VMEM((CHUNK, D_IN), jnp.float32),
        pltpu.SemaphoreType.DMA,
        pltpu.SemaphoreType.DMA,
        pltpu.SemaphoreType.DMA,
        pltpu.SemaphoreType.DMA,
        pltpu.SemaphoreType.DMA,
    ],
    name="hetero_gnn_scx",
)


def _sce_body(dst0, ef0, dst1, ef1, dst2, ef2,
              se_out, acce, dstb0, dstb1, efb16_0, efb16_1, efb0, efb1,
              seme0, seme1, sem):
    c = lax.axis_index("c")
    s = lax.axis_index("s")
    wid = c * NS + s
    zero16 = jnp.zeros((16,), jnp.float32)
    r0 = s * RPT
    o0 = c * NPAD + s * RPT

    def _one_type(m, dst_h, ef_h):
        def _fill(i, _):
            for j in range(D_IN // 16):
                efb0[i, pl.ds(j * 16, 16)] = zero16
                efb1[i, pl.ds(j * 16, 16)] = zero16
            return 0

        lax.fori_loop(0, ZROWS, _fill, 0)

        @pl.loop(0, ZITER)
        def _zero(k):
            pltpu.async_copy(efb0, acce.at[pl.ds(r0 + k * ZROWS, ZROWS)],
                             sem)

        @pl.loop(0, ZITER)
        def _zero_drain(k):
            pltpu.make_async_copy(
                efb0, acce.at[pl.ds(r0, ZROWS)], sem).wait()

        plsc.subcore_barrier()

        base0 = wid * EW

        def _load(j, dstb, efb16, seme):
            base = jnp.minimum(base0 + j * CHUNK, E - CHUNK)
            pltpu.async_copy(ef_h.at[pl.ds(base, CHUNK)], efb16, seme)
            pltpu.async_copy(dst_h.at[pl.ds(base, CHUNK)], dstb, seme)

        def _wait_load(dstb, efb16, seme):
            pltpu.make_async_copy(ef_h.at[pl.ds(0, CHUNK)], efb16, seme).wait()
            pltpu.make_async_copy(dst_h.at[pl.ds(0, CHUNK)], dstb, seme).wait()

        def _finish(dstb, efb16, efb, seme):
            _wait_load(dstb, efb16, seme)

            def _widen(i, _):
                efb[i, pl.ds(0, 16)] = efb16[i, :]
                return 0

            lax.fori_loop(0, CHUNK, _widen, 0, unroll=8)
            pltpu.async_copy(efb, acce.at[dstb], sem, add=True).wait()

        _load(0, dstb0, efb16_0, seme0)
        _load(1, dstb1, efb16_1, seme1)

        @pl.loop(0, NPAIR)
        def _pair(p):
            _finish(dstb0, efb16_0, efb0, seme0)
            _load(2 * p + 2, dstb0, efb16_0, seme0)
            _finish(dstb1, efb16_1, efb1, seme1)
            _load(2 * p + 3, dstb1, efb16_1, seme1)

        _finish(dstb0, efb16_0, efb0, seme0)
        _wait_load(dstb1, efb16_1, seme1)
        plsc.subcore_barrier()

        @pl.loop(0, ZITER // 2)
        def _dump(q):
            ra, rb = r0 + 2 * q * ZROWS, r0 + (2 * q + 1) * ZROWS
            oa, ob = o0 + 2 * q * ZROWS, o0 + (2 * q + 1) * ZROWS
            pltpu.async_copy(acce.at[pl.ds(ra, ZROWS)], efb0, seme0)
            pltpu.async_copy(acce.at[pl.ds(rb, ZROWS)], efb1, seme1)
            pltpu.make_async_copy(
                acce.at[pl.ds(ra, ZROWS)], efb0, seme0).wait()
            pltpu.async_copy(efb0, se_out.at[m, pl.ds(oa, ZROWS)], sem)
            pltpu.make_async_copy(
                acce.at[pl.ds(rb, ZROWS)], efb1, seme1).wait()
            pltpu.async_copy(efb1, se_out.at[m, pl.ds(ob, ZROWS)], sem)
            pltpu.make_async_copy(
                efb0, se_out.at[m, pl.ds(oa, ZROWS)], sem).wait()
            pltpu.make_async_copy(
                efb1, se_out.at[m, pl.ds(ob, ZROWS)], sem).wait()

        plsc.subcore_barrier()

    _one_type(0, dst0, ef0)
    _one_type(1, dst1, ef1)
    _one_type(2, dst2, ef2)


_sce_scatter = pl.kernel(
    _sce_body,
    out_type=jax.ShapeDtypeStruct((3, NC * NPAD, D_IN), jnp.float32),
    mesh=plsc.VectorSubcoreMesh(core_axis_name="c", subcore_axis_name="s"),
    scratch_types=[
        pltpu.VMEM_SHARED((NPAD, D_IN), jnp.float32),
        pltpu.VMEM((CHUNK,), jnp.int32),
        pltpu.VMEM((CHUNK,), jnp.int32),
        pltpu.VMEM((CHUNK, D_EDGE), jnp.float32),
        pltpu.VMEM((CHUNK, D_EDGE), jnp.float32),
        pltpu.VMEM((CHUNK, D_IN), jnp.float32),
        pltpu.VMEM((CHUNK, D_IN), jnp.float32),
        pltpu.SemaphoreType.DMA,
        pltpu.SemaphoreType.DMA,
        pltpu.SemaphoreType.DMA,
    ],
    name="hetero_gnn_sce",
)


def _tc_body(sx, se, xa, xp, wn, we, ws, ba, bp, outa, outp):
    f32 = jnp.float32
    acc = jnp.dot(sx[0, 0] + sx[0, 1], wn[0], preferred_element_type=f32)
    acc += jnp.dot(sx[1, 0] + sx[1, 1], wn[1], preferred_element_type=f32)
    acc += jnp.dot(se[0, 0, :, :D_EDGE] + se[0, 1, :, :D_EDGE], we[0],
                   preferred_element_type=f32)
    acc += jnp.dot(se[1, 0, :, :D_EDGE] + se[1, 1, :, :D_EDGE], we[1],
                   preferred_element_type=f32)
    acc += jnp.dot(xp[...], ws[0] + ws[1], preferred_element_type=f32)
    outp[...] = acc + bp[...]
    acc = jnp.dot(sx[2, 0] + sx[2, 1], wn[2], preferred_element_type=f32)
    acc += jnp.dot(se[2, 0, :, :D_EDGE] + se[2, 1, :, :D_EDGE], we[2],
                   preferred_element_type=f32)
    acc += jnp.dot(xa[...], ws[2], preferred_element_type=f32)
    outa[...] = acc + ba[...]


_BR = 1000


def _tc_combine(sx, se, xa, xp, wn, we, ws, ba, bp):
    nb = N_NODE // _BR
    return pl.pallas_call(
        _tc_body,
        grid=(nb,),
        in_specs=[
            pl.BlockSpec((3, NC, _BR, D_IN), lambda i: (0, 0, i, 0)),
            pl.BlockSpec((3, NC, _BR, D_IN), lambda i: (0, 0, i, 0)),
            pl.BlockSpec((_BR, D_IN), lambda i: (i, 0)),
            pl.BlockSpec((_BR, D_IN), lambda i: (i, 0)),
            pl.BlockSpec((3, D_IN, D_OUT), lambda i: (0, 0, 0)),
            pl.BlockSpec((3, D_EDGE, D_OUT), lambda i: (0, 0, 0)),
            pl.BlockSpec((3, D_IN, D_OUT), lambda i: (0, 0, 0)),
            pl.BlockSpec((1, D_OUT), lambda i: (0, 0)),
            pl.BlockSpec((1, D_OUT), lambda i: (0, 0)),
        ],
        out_specs=[
            pl.BlockSpec((_BR, D_OUT), lambda i: (i, 0)),
            pl.BlockSpec((_BR, D_OUT), lambda i: (i, 0)),
        ],
        out_shape=[
            jax.ShapeDtypeStruct((N_NODE, D_OUT), jnp.float32),
            jax.ShapeDtypeStruct((N_NODE, D_OUT), jnp.float32),
        ],
        name="hetero_gnn_tc_combine",
    )(sx, se, xa, xp, wn, we, ws, ba, bp)


def kernel(x_author, x_paper, edge_index_m0, edge_index_m1, edge_index_m2,
           edge_feat_m0, edge_feat_m1, edge_feat_m2,
           Wn0, Ws0, We0, b0, Wn1, Ws1, We1, b1, Wn2, Ws2, We2, b2):
    i32 = jnp.int32
    src0, dst0 = edge_index_m0[0].astype(i32), edge_index_m0[1].astype(i32)
    src1, dst1 = edge_index_m1[0].astype(i32), edge_index_m1[1].astype(i32)
    src2, dst2 = edge_index_m2[0].astype(i32), edge_index_m2[1].astype(i32)

    sx_p = _scx_scatter(x_author, x_paper, src0, dst0, src1, dst1, src2, dst2)
    se_p = _sce_scatter(dst0, edge_feat_m0, dst1, edge_feat_m1,
                        dst2, edge_feat_m2)
    sx = sx_p.reshape(3, NC, NPAD, D_IN)
    se = se_p.reshape(3, NC, NPAD, D_IN)

    wn = jnp.stack([Wn0, Wn1, Wn2])
    we = jnp.stack([We0, We1, We2])
    ws = jnp.stack([Ws0, Ws1, Ws2])
    ba = b2.reshape(1, D_OUT)
    bp = (b0 + b1).reshape(1, D_OUT)

    emb_author, emb_paper = _tc_combine(sx, se, x_author, x_paper,
                                        wn, we, ws, ba, bp)
    return (emb_author, emb_paper)
